# baseline (speedup 1.0000x reference)
import functools

import jax
import jax.numpy as jnp
from jax import lax
from jax.experimental import pallas as pl
from jax.experimental.pallas import tpu as pltpu
from jax.experimental.pallas import tpu_sc as plsc

B = 1024
D = 128
K = 32768
TEMP_INV = 1.0 / 0.07
BK = 2048
G = K // BK


def _row_normalize(x):
    ss = jnp.sum(x * x, axis=1, keepdims=True)
    return x * lax.rsqrt(jnp.maximum(ss, 1e-24))


def _sim_argmax_body(pr0_ref, pr1_ref, queue_ref, qid_ref, ids_ref,
                     idx_ref, qt_ref,
                     prn0_ref, prn1_ref, av0_ref, av1_ref,
                     rmax0_ref, rblk0_ref, rmax1_ref, rblk1_ref, ids2_ref):
    j = pl.program_id(0)

    @pl.when(j == 0)
    def _init():
        prn0_ref[...] = _row_normalize(pr0_ref[...])
        prn1_ref[...] = _row_normalize(pr1_ref[...])
        ids2_ref[...] = ids_ref[...].reshape(B, 1)

    qblk = queue_ref[...]
    qid = qid_ref[pl.ds(j * BK, BK)].reshape(1, BK)
    own = qid == ids2_ref[...]
    jf = jnp.float32(1.0) * j

    r = lax.broadcasted_iota(jnp.int32, (D, D), 0)
    c = lax.broadcasted_iota(jnp.int32, (D, D), 1)
    eye = jnp.where(r == c, 1.0, 0.0).astype(jnp.float32)
    qt_ref[...] = lax.dot_general(qblk, eye, (((0,), (0,)), ((), ())),
                                  preferred_element_type=jnp.float32)

    def one_view(prn_ref, av_ref, rmax_ref, rblk_ref):
        sim = lax.dot_general(prn_ref[...], qblk, (((1,), (0,)), ((), ())),
                              preferred_element_type=jnp.float32)
        sim = jnp.where(own, -1.0, sim)
        bmaxj = jnp.max(sim, axis=1, keepdims=True)

        @pl.when(j == 0)
        def _first():
            av_ref[...] = sim
            rmax_ref[...] = bmaxj
            rblk_ref[...] = jnp.zeros((B, 1), jnp.float32)

        @pl.when(j > 0)
        def _rest():
            av_ref[...] = jnp.maximum(av_ref[...], sim)
            better = bmaxj > rmax_ref[...]
            rmax_ref[...] = jnp.where(better, bmaxj, rmax_ref[...])
            rblk_ref[...] = jnp.where(better, jf, rblk_ref[...])

    one_view(prn0_ref, av0_ref, rmax0_ref, rblk0_ref)
    one_view(prn1_ref, av1_ref, rmax1_ref, rblk1_ref)

    @pl.when(j == G - 1)
    def _fin():
        colf = lax.broadcasted_iota(jnp.int32, (B, BK), 1).astype(jnp.float32)

        def extract(av_ref, rmax_ref, rblk_ref):
            cand = jnp.where(av_ref[...] == rmax_ref[...], colf,
                             jnp.float32(1e9))
            lmin = jnp.min(cand, axis=1, keepdims=True)
            return (rblk_ref[...] * float(BK) + lmin).astype(jnp.int32)

        idx_ref[pl.ds(0, B)] = extract(av0_ref, rmax0_ref, rblk0_ref).reshape(B)
        idx_ref[pl.ds(B, B)] = extract(av1_ref, rmax1_ref, rblk1_ref).reshape(B)


def _sim_argmax(pr0, pr1, queue, qid2, ids2):
    return pl.pallas_call(
        _sim_argmax_body,
        grid=(G,),
        in_specs=[
            pl.BlockSpec((B, D), lambda j: (0, 0)),
            pl.BlockSpec((B, D), lambda j: (0, 0)),
            pl.BlockSpec((D, BK), lambda j: (0, j)),
            pl.BlockSpec((K,), lambda j: (0,)),
            pl.BlockSpec((B,), lambda j: (0,)),
        ],
        out_specs=[
            pl.BlockSpec((2 * B,), lambda j: (0,)),
            pl.BlockSpec((BK, D), lambda j: (j, 0)),
        ],
        out_shape=[
            jax.ShapeDtypeStruct((2 * B,), jnp.int32),
            jax.ShapeDtypeStruct((K, D), jnp.float32),
        ],
        scratch_shapes=[
            pltpu.VMEM((B, D), jnp.float32),
            pltpu.VMEM((B, D), jnp.float32),
            pltpu.VMEM((B, BK), jnp.float32),
            pltpu.VMEM((B, BK), jnp.float32),
            pltpu.VMEM((B, 1), jnp.float32),
            pltpu.VMEM((B, 1), jnp.float32),
            pltpu.VMEM((B, 1), jnp.float32),
            pltpu.VMEM((B, 1), jnp.float32),
            pltpu.VMEM((B, 1), jnp.int32),
        ],
    )(pr0, pr1, queue, qid2, ids2)


@functools.lru_cache(maxsize=None)
def _make_sc_gather(nrows):
    info = plsc.get_sparse_core_info()
    nc, ns = info.num_cores, info.num_subcores
    nw = nc * ns
    per_w = nrows // nw
    mesh = plsc.VectorSubcoreMesh(core_axis_name="c", subcore_axis_name="s")

    @functools.partial(
        pl.kernel,
        out_type=jax.ShapeDtypeStruct((nrows, D), jnp.float32),
        mesh=mesh,
        scratch_types=[
            pltpu.VMEM((per_w,), jnp.int32),
            pltpu.VMEM((per_w, D), jnp.float32),
            pltpu.SemaphoreType.DMA,
        ],
    )
    def gather(table_hbm, idx_hbm, out_hbm, idx_v, rows_v, sem):
        wid = lax.axis_index("s") * nc + lax.axis_index("c")
        base = wid * per_w
        pltpu.sync_copy(idx_hbm.at[pl.ds(base, per_w)], idx_v)
        pltpu.async_copy(table_hbm.at[idx_v], rows_v, sem).wait()
        pltpu.sync_copy(rows_v, out_hbm.at[pl.ds(base, per_w)])

    return gather


def _loss_body(p0_ref, p1_ref, nn_ref, out_ref):
    p0n = _row_normalize(p0_ref[...])
    p1n = _row_normalize(p1_ref[...])
    r = lax.broadcasted_iota(jnp.int32, (B, B), 0)
    c = lax.broadcasted_iota(jnp.int32, (B, B), 1)

    def one_loss(nn, pn):
        logits = lax.dot_general(nn, pn, (((1,), (1,)), ((), ())),
                                 preferred_element_type=jnp.float32)
        logits = logits * TEMP_INV
        m = jnp.max(logits, axis=1, keepdims=True)
        lse = m + jnp.log(jnp.sum(jnp.exp(logits - m), axis=1, keepdims=True))
        diag = jnp.sum(jnp.where(r == c, logits, 0.0), axis=1, keepdims=True)
        return jnp.mean(lse - diag)

    out_ref[0, 0] = 0.5 * (one_loss(nn_ref[B:2 * B, :], p0n)
                           + one_loss(nn_ref[0:B, :], p1n))


def _loss(p0, p1, nn):
    return pl.pallas_call(
        _loss_body,
        out_specs=pl.BlockSpec(memory_space=pltpu.SMEM),
        out_shape=jax.ShapeDtypeStruct((1, 1), jnp.float32),
    )(p0, p1, nn)


def kernel(predict0, predict1, project0, project1, queue, queue_id, ids):
    idx, qt = _sim_argmax(project0, project1, queue, queue_id, ids)
    nn = _make_sc_gather(2 * B)(qt, idx)
    loss = _loss(predict0, predict1, nn)
    return loss[0, 0]


# speedup vs baseline: 1.3595x; 1.3595x over previous
import functools

import jax
import jax.numpy as jnp
from jax import lax
from jax.experimental import pallas as pl
from jax.experimental.pallas import tpu as pltpu
from jax.experimental.pallas import tpu_sc as plsc

B = 1024
D = 128
K = 32768
TEMP_INV = 1.0 / 0.07
BK = 2048
G = K // BK


def _row_normalize(x):
    ss = jnp.sum(x * x, axis=1, keepdims=True)
    return x * lax.rsqrt(jnp.maximum(ss, 1e-24))


def _sim_argmax_body(pr0_ref, pr1_ref, queue_ref, qid_ref, ids_ref,
                     idx_ref, qt_ref,
                     prn0_ref, prn1_ref, av0_ref, av1_ref,
                     rmax0_ref, rblk0_ref, rmax1_ref, rblk1_ref, ids2_ref):
    j = pl.program_id(0)

    @pl.when(j == 0)
    def _init():
        prn0_ref[...] = _row_normalize(pr0_ref[...])
        prn1_ref[...] = _row_normalize(pr1_ref[...])
        av0_ref[...] = jnp.full((B, BK), -jnp.inf, jnp.float32)
        av1_ref[...] = jnp.full((B, BK), -jnp.inf, jnp.float32)
        rmax0_ref[...] = jnp.full((B, 1), -jnp.inf, jnp.float32)
        rmax1_ref[...] = jnp.full((B, 1), -jnp.inf, jnp.float32)
        rblk0_ref[...] = jnp.zeros((B, 1), jnp.float32)
        rblk1_ref[...] = jnp.zeros((B, 1), jnp.float32)
        ids2_ref[...] = ids_ref[...].reshape(B, 1)

    qblk = queue_ref[...]
    qid = qid_ref[pl.ds(j * BK, BK)].reshape(1, BK)
    own = qid == ids2_ref[...]
    jf = jnp.float32(1.0) * j

    r = lax.broadcasted_iota(jnp.int32, (D, D), 0)
    c = lax.broadcasted_iota(jnp.int32, (D, D), 1)
    eye = jnp.where(r == c, 1.0, 0.0).astype(jnp.float32)
    qt_ref[...] = lax.dot_general(qblk, eye, (((0,), (0,)), ((), ())),
                                  preferred_element_type=jnp.float32)

    def one_view(prn_ref, av_ref, rmax_ref, rblk_ref):
        sim = lax.dot_general(prn_ref[...], qblk, (((1,), (0,)), ((), ())),
                              preferred_element_type=jnp.float32)
        sim = jnp.where(own, -1.0, sim)
        av_ref[...] = jnp.maximum(av_ref[...], sim)
        bmaxj = jnp.max(sim, axis=1, keepdims=True)
        better = bmaxj > rmax_ref[...]
        rmax_ref[...] = jnp.where(better, bmaxj, rmax_ref[...])
        rblk_ref[...] = jnp.where(better, jf, rblk_ref[...])

    one_view(prn0_ref, av0_ref, rmax0_ref, rblk0_ref)
    one_view(prn1_ref, av1_ref, rmax1_ref, rblk1_ref)

    @pl.when(j == G - 1)
    def _fin():
        colf = lax.broadcasted_iota(jnp.int32, (B, BK), 1).astype(jnp.float32)

        def extract(av_ref, rmax_ref, rblk_ref):
            cand = jnp.where(av_ref[...] == rmax_ref[...], colf,
                             jnp.float32(1e9))
            lmin = jnp.min(cand, axis=1, keepdims=True)
            return (rblk_ref[...] * float(BK) + lmin).astype(jnp.int32)

        idx_ref[pl.ds(0, B)] = extract(av0_ref, rmax0_ref, rblk0_ref).reshape(B)
        idx_ref[pl.ds(B, B)] = extract(av1_ref, rmax1_ref, rblk1_ref).reshape(B)


def _sim_argmax(pr0, pr1, queue, qid2, ids2):
    return pl.pallas_call(
        _sim_argmax_body,
        grid=(G,),
        in_specs=[
            pl.BlockSpec((B, D), lambda j: (0, 0)),
            pl.BlockSpec((B, D), lambda j: (0, 0)),
            pl.BlockSpec((D, BK), lambda j: (0, j)),
            pl.BlockSpec((K,), lambda j: (0,)),
            pl.BlockSpec((B,), lambda j: (0,)),
        ],
        out_specs=[
            pl.BlockSpec((2 * B,), lambda j: (0,)),
            pl.BlockSpec((BK, D), lambda j: (j, 0)),
        ],
        out_shape=[
            jax.ShapeDtypeStruct((2 * B,), jnp.int32),
            jax.ShapeDtypeStruct((K, D), jnp.float32),
        ],
        scratch_shapes=[
            pltpu.VMEM((B, D), jnp.float32),
            pltpu.VMEM((B, D), jnp.float32),
            pltpu.VMEM((B, BK), jnp.float32),
            pltpu.VMEM((B, BK), jnp.float32),
            pltpu.VMEM((B, 1), jnp.float32),
            pltpu.VMEM((B, 1), jnp.float32),
            pltpu.VMEM((B, 1), jnp.float32),
            pltpu.VMEM((B, 1), jnp.float32),
            pltpu.VMEM((B, 1), jnp.int32),
        ],
    )(pr0, pr1, queue, qid2, ids2)


@functools.lru_cache(maxsize=None)
def _make_sc_gather(nrows):
    info = plsc.get_sparse_core_info()
    nc, ns = info.num_cores, info.num_subcores
    nw = nc * ns
    per_w = nrows // nw
    mesh = plsc.VectorSubcoreMesh(core_axis_name="c", subcore_axis_name="s")

    @functools.partial(
        pl.kernel,
        out_type=jax.ShapeDtypeStruct((nrows, D), jnp.float32),
        mesh=mesh,
        scratch_types=[
            pltpu.VMEM((per_w,), jnp.int32),
            pltpu.VMEM((per_w, D), jnp.float32),
            pltpu.SemaphoreType.DMA,
        ],
    )
    def gather(table_hbm, idx_hbm, out_hbm, idx_v, rows_v, sem):
        wid = lax.axis_index("s") * nc + lax.axis_index("c")
        base = wid * per_w
        pltpu.sync_copy(idx_hbm.at[pl.ds(base, per_w)], idx_v)
        pltpu.async_copy(table_hbm.at[idx_v], rows_v, sem).wait()
        pltpu.sync_copy(rows_v, out_hbm.at[pl.ds(base, per_w)])

    return gather


def _loss_body(p0_ref, p1_ref, nn_ref, out_ref):
    p0n = _row_normalize(p0_ref[...])
    p1n = _row_normalize(p1_ref[...])
    r = lax.broadcasted_iota(jnp.int32, (B, B), 0)
    c = lax.broadcasted_iota(jnp.int32, (B, B), 1)

    def one_loss(nn, pn):
        logits = lax.dot_general(nn, pn, (((1,), (1,)), ((), ())),
                                 preferred_element_type=jnp.float32)
        logits = logits * TEMP_INV
        m = jnp.max(logits, axis=1, keepdims=True)
        lse = m + jnp.log(jnp.sum(jnp.exp(logits - m), axis=1, keepdims=True))
        diag = jnp.sum(jnp.where(r == c, logits, 0.0), axis=1, keepdims=True)
        return jnp.mean(lse - diag)

    out_ref[0, 0] = 0.5 * (one_loss(nn_ref[B:2 * B, :], p0n)
                           + one_loss(nn_ref[0:B, :], p1n))


def _loss(p0, p1, nn):
    return pl.pallas_call(
        _loss_body,
        out_specs=pl.BlockSpec(memory_space=pltpu.SMEM),
        out_shape=jax.ShapeDtypeStruct((1, 1), jnp.float32),
    )(p0, p1, nn)


def kernel(predict0, predict1, project0, project1, queue, queue_id, ids):
    idx, qt = _sim_argmax(project0, project1, queue, queue_id, ids)
    nn = _make_sc_gather(2 * B)(qt, idx)
    loss = _loss(predict0, predict1, nn)
    return loss[0, 0]
